# Initial kernel scaffold; baseline (speedup 1.0000x reference)
import jax
import jax.numpy as jnp
from jax.experimental import pallas as pl


def kernel(fish_input, pv_input, front_input, uu, vv, valid, density):
    raise NotImplementedError("write your pallas kernel here")


# speedup vs baseline: 1.5573x; 1.5573x over previous
import functools

import jax
import jax.numpy as jnp
from jax import lax
from jax.experimental import pallas as pl
from jax.experimental.pallas import tpu as pltpu
from jax.experimental.pallas import tpu_sc as plsc

L = 172800
C = 336
H, W = 128, 224
BH, BW = 240, 120
D = 6
CELLS = BH * BW
PLANE = H * W
NROWS = D * C
NWORKERS = 32
ROWS_PER_W = NROWS // NWORKERS
CHUNK = 7200
UNROLL = 10
VPC = CHUNK // 16


def _body(fish_ref, pv_ref, front_ref, p_ref, validf_ref, dens_ref, out_ref,
          idx_v, w_v, plane_v, out_v):
    cid = lax.axis_index("c")
    sid = lax.axis_index("s")
    wid = sid * 2 + cid
    q0 = wid * ROWS_PER_W

    plane = plane_v.at[pl.ds(0, PLANE)]

    def prep(d):
        seg = pl.ds(d * CELLS, CELLS)
        pltpu.sync_copy(p_ref.at[seg], idx_v)
        pltpu.sync_copy(dens_ref.at[seg], w_v)
        pltpu.sync_copy(validf_ref.at[seg], plane_v.at[pl.ds(0, CELLS)])

        def wstep(j, _):
            s = pl.ds(j * 16, 16)
            w_v[s] = w_v[s] * plane_v[s]
            return 0

        lax.fori_loop(0, CELLS // 16, wstep, 0)

    def row(q, _):
        d = q // C
        c = q % C

        @pl.when(jnp.logical_or(q == q0, c == 0))
        def _():
            prep(d)

        cam = d // 2
        csrc = pl.ds(c * PLANE, PLANE)

        @pl.when(cam == 0)
        def _():
            pltpu.sync_copy(fish_ref.at[csrc], plane)

        @pl.when(cam == 1)
        def _():
            pltpu.sync_copy(pv_ref.at[csrc], plane)

        @pl.when(cam == 2)
        def _():
            pltpu.sync_copy(front_ref.at[csrc], plane)

        for k in range(CELLS // CHUNK):
            obuf = pl.ds((k % 2) * CHUNK, CHUNK)

            def gstep(g, _):
                for u in range(UNROLL):
                    t = k * CHUNK + (g * UNROLL + u) * 16
                    s = pl.ds(t, 16)
                    idx = idx_v[s]
                    vals = plsc.load_gather(plane, [idx])
                    out_v[pl.ds((k % 2) * CHUNK + (g * UNROLL + u) * 16, 16)] = (
                        vals * w_v[s])
                return 0

            lax.fori_loop(0, VPC // UNROLL, gstep, 0)
            pltpu.sync_copy(out_v.at[obuf],
                            out_ref.at[pl.ds(q * CELLS + k * CHUNK, CHUNK)])
        return 0

    lax.fori_loop(q0, q0 + ROWS_PER_W, row, 0)


@jax.jit
def _run(fish_i, pv_i, front_i, p, valid_f, dens):
    mesh = plsc.VectorSubcoreMesh(core_axis_name="c", subcore_axis_name="s",
                                  num_cores=2, num_subcores=16)
    return pl.kernel(
        _body,
        out_type=jax.ShapeDtypeStruct((NROWS * CELLS,), jnp.float32),
        mesh=mesh,
        compiler_params=pltpu.CompilerParams(needs_layout_passes=False),
        scratch_types=[
            pltpu.VMEM((CELLS,), jnp.int32),
            pltpu.VMEM((CELLS,), jnp.float32),
            pltpu.VMEM((2 * PLANE,), jnp.float32),
            pltpu.VMEM((2 * CHUNK,), jnp.float32),
        ],
    )(fish_i, pv_i, front_i, p, valid_f, dens)


def kernel(fish_input, pv_input, front_input, uu, vv, valid, density):
    fish_i = fish_input.reshape(C * PLANE)
    pv_i = pv_input.reshape(C * PLANE)
    front_i = front_input.reshape(C * PLANE)
    p = vv * W + uu
    valid_f = valid.astype(jnp.float32)
    out = _run(fish_i, pv_i, front_i, p, valid_f, density)
    return out.reshape(1, NROWS, BH, BW)


# speedup vs baseline: 1.6993x; 1.0912x over previous
import functools

import jax
import jax.numpy as jnp
from jax import lax
from jax.experimental import pallas as pl
from jax.experimental.pallas import tpu as pltpu
from jax.experimental.pallas import tpu_sc as plsc

L = 172800
C = 336
H, W = 128, 224
BH, BW = 240, 120
D = 6
CELLS = BH * BW
PLANE = H * W
NROWS = D * C
NWORKERS = 32
ROWS_PER_W = NROWS // NWORKERS
CHUNK = 7200
UNROLL = 10
VPC = CHUNK // 16


def _body(fish_ref, pv_ref, front_ref, p_ref, valid_ref, dens_ref, out_ref,
          idx_v, w_v, plane_v, out_v, psem, osem0, osem1):
    osems = (osem0, osem1)
    cid = lax.axis_index("c")
    sid = lax.axis_index("s")
    wid = sid * 2 + cid
    q0 = wid * ROWS_PER_W

    def start_plane(q):
        d = q // C
        c = q % C
        cam = d // 2
        csrc = pl.ds(c * PLANE, PLANE)
        dst = plane_v.at[pl.ds((q % 2) * PLANE, PLANE)]

        @pl.when(cam == 0)
        def _():
            pltpu.async_copy(fish_ref.at[csrc], dst, psem)

        @pl.when(cam == 1)
        def _():
            pltpu.async_copy(pv_ref.at[csrc], dst, psem)

        @pl.when(cam == 2)
        def _():
            pltpu.async_copy(front_ref.at[csrc], dst, psem)

    def wait_plane(q):
        pltpu.make_async_copy(
            fish_ref.at[pl.ds(0, PLANE)],
            plane_v.at[pl.ds((q % 2) * PLANE, PLANE)], psem).wait()

    def drain_out(h):
        pltpu.make_async_copy(out_v.at[pl.ds(0, CHUNK)],
                              out_ref.at[pl.ds(0, CHUNK)], osems[h]).wait()

    def prep(d):
        seg = pl.ds(d * CELLS, CELLS)
        pltpu.sync_copy(dens_ref.at[seg], w_v)
        pltpu.sync_copy(valid_ref.at[seg], idx_v)

        def wstep(j, _):
            s = pl.ds(j * 16, 16)
            w_v[s] = w_v[s] * idx_v[s].astype(jnp.float32)
            return 0

        lax.fori_loop(0, CELLS // 16, wstep, 0)
        pltpu.sync_copy(p_ref.at[seg], idx_v)

    def row(q, _):
        d = q // C
        c = q % C

        @pl.when(jnp.logical_or(q == q0, c == 0))
        def _():
            prep(d)

        wait_plane(q)

        @pl.when(q + 1 < q0 + ROWS_PER_W)
        def _():
            start_plane(q + 1)

        plane = plane_v.at[pl.ds((q % 2) * PLANE, PLANE)]

        for k in range(CELLS // CHUNK):
            if k >= 2:
                drain_out(k % 2)
            else:
                @pl.when(q > q0)
                def _():
                    drain_out(k % 2)

            def gstep(g, _):
                for u in range(UNROLL):
                    t = (g * UNROLL + u) * 16
                    s = pl.ds(k * CHUNK + t, 16)
                    idx = idx_v[s]
                    vals = plsc.load_gather(plane, [idx])
                    out_v[pl.ds((k % 2) * CHUNK + t, 16)] = vals * w_v[s]
                return 0

            lax.fori_loop(0, VPC // UNROLL, gstep, 0)
            pltpu.async_copy(out_v.at[pl.ds((k % 2) * CHUNK, CHUNK)],
                             out_ref.at[pl.ds(q * CELLS + k * CHUNK, CHUNK)],
                             osems[k % 2])
        return 0

    start_plane(q0)
    lax.fori_loop(q0, q0 + ROWS_PER_W, row, 0)
    drain_out(0)
    drain_out(1)


@jax.jit
def _run(fish_i, pv_i, front_i, p, valid, dens):
    mesh = plsc.VectorSubcoreMesh(core_axis_name="c", subcore_axis_name="s",
                                  num_cores=2, num_subcores=16)
    return pl.kernel(
        _body,
        out_type=jax.ShapeDtypeStruct((NROWS * CELLS,), jnp.float32),
        mesh=mesh,
        compiler_params=pltpu.CompilerParams(needs_layout_passes=False),
        scratch_types=[
            pltpu.VMEM((CELLS,), jnp.int32),
            pltpu.VMEM((CELLS,), jnp.float32),
            pltpu.VMEM((2 * PLANE,), jnp.float32),
            pltpu.VMEM((2 * CHUNK,), jnp.float32),
            pltpu.SemaphoreType.DMA,
            pltpu.SemaphoreType.DMA,
            pltpu.SemaphoreType.DMA,
        ],
    )(fish_i, pv_i, front_i, p, valid, dens)


def kernel(fish_input, pv_input, front_input, uu, vv, valid, density):
    fish_i = fish_input.reshape(C * PLANE)
    pv_i = pv_input.reshape(C * PLANE)
    front_i = front_input.reshape(C * PLANE)
    p = vv * W + uu
    out = _run(fish_i, pv_i, front_i, p, valid, density)
    return out.reshape(1, NROWS, BH, BW)


# speedup vs baseline: 2.7732x; 1.6320x over previous
import functools

import jax
import jax.numpy as jnp
from jax import lax
from jax.experimental import pallas as pl
from jax.experimental.pallas import tpu as pltpu
from jax.experimental.pallas import tpu_sc as plsc

L = 172800
C = 336
H, W = 128, 224
BH, BW = 240, 120
D = 6
CELLS = BH * BW
PLANE = H * W
NROWS = D * C
NWORKERS = 32
ROWS_PER_W = NROWS // NWORKERS
CHUNK = 7200
UNROLL = 10
VPC = CHUNK // 16


def _body(fish_ref, pv_ref, front_ref, p_ref, valid_ref, dens_ref, out_ref,
          idx_v, w_v, plane_v, out_v, psem, osem0, osem1):
    osems = (osem0, osem1)
    cid = lax.axis_index("c")
    sid = lax.axis_index("s")
    wid = sid * 2 + cid
    q0 = wid * ROWS_PER_W

    def start_plane(q):
        d = q // C
        c = q % C
        cam = d // 2
        csrc = pl.ds(c * PLANE, PLANE)
        dst = plane_v.at[pl.ds((q % 2) * PLANE, PLANE)]

        @pl.when(cam == 0)
        def _():
            pltpu.async_copy(fish_ref.at[csrc], dst, psem)

        @pl.when(cam == 1)
        def _():
            pltpu.async_copy(pv_ref.at[csrc], dst, psem)

        @pl.when(cam == 2)
        def _():
            pltpu.async_copy(front_ref.at[csrc], dst, psem)

    def wait_plane(q):
        pltpu.make_async_copy(
            fish_ref.at[pl.ds(0, PLANE)],
            plane_v.at[pl.ds((q % 2) * PLANE, PLANE)], psem).wait()

    def drain_out(h):
        pltpu.make_async_copy(out_v.at[pl.ds(0, CHUNK)],
                              out_ref.at[pl.ds(0, CHUNK)], osems[h]).wait()

    def prep(d):
        seg = pl.ds(d * CELLS, CELLS)
        pltpu.sync_copy(dens_ref.at[seg], w_v)
        pltpu.sync_copy(valid_ref.at[seg], idx_v)

        def wstep(j, _):
            s = pl.ds(j * 16, 16)
            w_v[s] = w_v[s] * idx_v[s].astype(jnp.float32)
            return 0

        lax.fori_loop(0, CELLS // 16, wstep, 0)
        pltpu.sync_copy(p_ref.at[seg], idx_v)

    def row(q, _):
        d = q // C
        c = q % C

        @pl.when(jnp.logical_or(q == q0, c == 0))
        def _():
            prep(d)

        wait_plane(q)

        @pl.when(q + 1 < q0 + ROWS_PER_W)
        def _():
            start_plane(q + 1)

        plane = plane_v.at[pl.ds((q % 2) * PLANE, PLANE)]

        for k in range(CELLS // CHUNK):
            if k >= 2:
                drain_out(k % 2)
            else:
                @pl.when(q > q0)
                def _():
                    drain_out(k % 2)

            @plsc.parallel_loop(0, VPC, 1, unroll=UNROLL)
            def gstep(g):
                t = g * 16
                s = pl.ds(k * CHUNK + t, 16)
                idx = idx_v[s]
                vals = plsc.load_gather(plane, [idx])
                out_v[pl.ds((k % 2) * CHUNK + t, 16)] = vals * w_v[s]
            pltpu.async_copy(out_v.at[pl.ds((k % 2) * CHUNK, CHUNK)],
                             out_ref.at[pl.ds(q * CELLS + k * CHUNK, CHUNK)],
                             osems[k % 2])
        return 0

    start_plane(q0)
    lax.fori_loop(q0, q0 + ROWS_PER_W, row, 0)
    drain_out(0)
    drain_out(1)


@jax.jit
def _run(fish_i, pv_i, front_i, p, valid, dens):
    mesh = plsc.VectorSubcoreMesh(core_axis_name="c", subcore_axis_name="s",
                                  num_cores=2, num_subcores=16)
    return pl.kernel(
        _body,
        out_type=jax.ShapeDtypeStruct((NROWS * CELLS,), jnp.float32),
        mesh=mesh,
        compiler_params=pltpu.CompilerParams(needs_layout_passes=False),
        scratch_types=[
            pltpu.VMEM((CELLS,), jnp.int32),
            pltpu.VMEM((CELLS,), jnp.float32),
            pltpu.VMEM((2 * PLANE,), jnp.float32),
            pltpu.VMEM((2 * CHUNK,), jnp.float32),
            pltpu.SemaphoreType.DMA,
            pltpu.SemaphoreType.DMA,
            pltpu.SemaphoreType.DMA,
        ],
    )(fish_i, pv_i, front_i, p, valid, dens)


def kernel(fish_input, pv_input, front_input, uu, vv, valid, density):
    fish_i = fish_input.reshape(C * PLANE)
    pv_i = pv_input.reshape(C * PLANE)
    front_i = front_input.reshape(C * PLANE)
    p = vv * W + uu
    out = _run(fish_i, pv_i, front_i, p, valid, density)
    return out.reshape(1, NROWS, BH, BW)
